# Initial kernel scaffold; baseline (speedup 1.0000x reference)
import jax
import jax.numpy as jnp
from jax.experimental import pallas as pl


def kernel(x, edge_index, W1_l, b1, W1_r, W2_l, b2, W2_r):
    raise NotImplementedError("write your pallas kernel here")


# speedup vs baseline: 5.3940x; 5.3940x over previous
import functools

import jax
import jax.numpy as jnp
from jax import lax
from jax.experimental import pallas as pl
from jax.experimental.pallas import tpu as pltpu
from jax.experimental.pallas import tpu_sc as plsc

N = 10000
E = 320000
F_IN = 128
HID = 128
CLS = 40

D1 = 144
D2 = 48

NC = 2
NS = 16
NW = NC * NS
E_PER_W = E // NW
CHUNK = 80
NCHUNK = E_PER_W // CHUNK
N_PAD = 10240
ROWS_PER_TILE = N_PAD // NS

ROW_BLK = 1000
GRID = N // ROW_BLK


def _make_seg_sum(D):
  mesh = plsc.VectorSubcoreMesh(core_axis_name="c", subcore_axis_name="s")

  @functools.partial(
      pl.kernel,
      mesh=mesh,
      compiler_params=pltpu.CompilerParams(use_tc_tiling_on_sc=False),
      out_type=jax.ShapeDtypeStruct((NC, N_PAD, D), jnp.float32),
      scratch_types=[
          pltpu.VMEM((CHUNK,), jnp.int32),
          pltpu.VMEM((CHUNK,), jnp.int32),
          pltpu.VMEM((CHUNK, D), jnp.float32),
          pltpu.VMEM_SHARED((N_PAD, D), jnp.float32),
          pltpu.SemaphoreType.DMA,
      ],
  )
  def seg_sum(p_hbm, src_hbm, dst_hbm, zeros_hbm, out_hbm,
              src_v, dst_v, rows_v, acc_sh, sem):
    c = lax.axis_index("c")
    s = lax.axis_index("s")
    wid = c * NS + s
    row0 = pl.multiple_of(s * ROWS_PER_TILE, 8)

    pltpu.sync_copy(zeros_hbm.at[pl.ds(row0, ROWS_PER_TILE)],
                    acc_sh.at[pl.ds(row0, ROWS_PER_TILE)])
    plsc.subcore_barrier()

    base = wid * E_PER_W

    def body(i, carry):
      off = pl.multiple_of(base + i * CHUNK, 8)
      pltpu.sync_copy(src_hbm.at[pl.ds(off, CHUNK)], src_v)
      pltpu.sync_copy(dst_hbm.at[pl.ds(off, CHUNK)], dst_v)
      pltpu.async_copy(p_hbm.at[src_v], rows_v, sem).wait()
      pltpu.sync_copy(rows_v, acc_sh.at[dst_v], add=True)
      return carry

    lax.fori_loop(0, NCHUNK, body, 0)
    plsc.subcore_barrier()

    pltpu.sync_copy(acc_sh.at[pl.ds(row0, ROWS_PER_TILE)],
                    out_hbm.at[c, pl.ds(row0, ROWS_PER_TILE)])

  return seg_sum


_seg_sum_1 = _make_seg_sum(D1)
_seg_sum_2 = _make_seg_sum(D2)


def _l1_body(x_ref, w1le_ref, w1rt_ref, b1_ref, p1e_ref, pre1_ref):
  xb = x_ref[...]
  deg_col = (lax.broadcasted_iota(jnp.int32, (ROW_BLK, D1), 1) == F_IN
             ).astype(jnp.float32)
  p1e_ref[...] = jnp.dot(xb, w1le_ref[...],
                         preferred_element_type=jnp.float32) + deg_col
  pre1_ref[...] = jnp.dot(xb, w1rt_ref[...],
                          preferred_element_type=jnp.float32) + b1_ref[...]


def _l2_body(a0_ref, a1_ref, pre1_ref, w2le_ref, w2rt_ref, b2_ref,
             p2e_ref, pre2_ref):
  acc = a0_ref[...] + a1_ref[...]
  agg = acc[:, :HID]
  deg = acc[:, HID:HID + 1]
  h = jnp.maximum(agg / jnp.maximum(deg, 1.0) + pre1_ref[...], 0.0)
  deg_col = (lax.broadcasted_iota(jnp.int32, (ROW_BLK, D2), 1) == CLS
             ).astype(jnp.float32)
  p2e_ref[...] = jnp.dot(h, w2le_ref[...],
                         preferred_element_type=jnp.float32) + deg_col
  pre2_ref[...] = jnp.dot(h, w2rt_ref[...],
                          preferred_element_type=jnp.float32) + b2_ref[...]


def _out_body(a0_ref, a1_ref, pre2_ref, out_ref):
  acc = a0_ref[...] + a1_ref[...]
  agg = acc[:, :CLS]
  deg = acc[:, CLS:CLS + 1]
  o = agg / jnp.maximum(deg, 1.0) + pre2_ref[...]
  m = jnp.max(o, axis=1, keepdims=True)
  lse = jnp.log(jnp.sum(jnp.exp(o - m), axis=1, keepdims=True))
  out_ref[...] = o - m - lse


def _row_spec(d):
  return pl.BlockSpec((ROW_BLK, d), lambda i: (i, 0))


def _full_spec(r, d):
  return pl.BlockSpec((r, d), lambda i: (0, 0))


_l1_call = pl.pallas_call(
    _l1_body,
    grid=(GRID,),
    in_specs=[_row_spec(F_IN), _full_spec(F_IN, D1), _full_spec(F_IN, HID),
              _full_spec(1, HID)],
    out_specs=[_row_spec(D1), _row_spec(HID)],
    out_shape=[jax.ShapeDtypeStruct((N, D1), jnp.float32),
               jax.ShapeDtypeStruct((N, HID), jnp.float32)],
)

_l2_call = pl.pallas_call(
    _l2_body,
    grid=(GRID,),
    in_specs=[_row_spec(D1), _row_spec(D1), _row_spec(HID),
              _full_spec(HID, D2), _full_spec(HID, CLS), _full_spec(1, CLS)],
    out_specs=[_row_spec(D2), _row_spec(CLS)],
    out_shape=[jax.ShapeDtypeStruct((N, D2), jnp.float32),
               jax.ShapeDtypeStruct((N, CLS), jnp.float32)],
)

_out_call = pl.pallas_call(
    _out_body,
    grid=(GRID,),
    in_specs=[_row_spec(D2), _row_spec(D2), _row_spec(CLS)],
    out_specs=_row_spec(CLS),
    out_shape=jax.ShapeDtypeStruct((N, CLS), jnp.float32),
)


def kernel(x, edge_index, W1_l, b1, W1_r, W2_l, b2, W2_r):
  src = edge_index[0]
  dst = edge_index[1]

  w1le = jnp.pad(W1_l.T, ((0, 0), (0, D1 - HID)))
  w1rt = W1_r.T
  w2le = jnp.pad(W2_l.T, ((0, 0), (0, D2 - CLS)))
  w2rt = W2_r.T

  p1e, pre1 = _l1_call(x, w1le, w1rt, b1.reshape(1, HID))
  agg1 = _seg_sum_1(p1e, src, dst, jnp.zeros((N_PAD, D1), jnp.float32))
  p2e, pre2 = _l2_call(agg1[0], agg1[1], pre1, w2le, w2rt,
                       b2.reshape(1, CLS))
  agg2 = _seg_sum_2(p2e, src, dst, jnp.zeros((N_PAD, D2), jnp.float32))
  return _out_call(agg2[0], agg2[1], pre2)
